# baseline (speedup 1.0000x reference)
import functools

import jax
import jax.numpy as jnp
from jax import lax
from jax.experimental import pallas as pl
from jax.experimental.pallas import tpu as pltpu
from jax.experimental.pallas import tpu_sc as plsc

_NUM_TABLES = 26
_VOCAB = 100000
_EMB = 32
_BATCH = 4096
_NCOL = _NUM_TABLES + 1
_NW = 32
_ROWS_PER_W = _BATCH * _NCOL // _NW
_CHUNK = 128
_NCHUNK = _ROWS_PER_W // _CHUNK
_LANES = 16


def _sc_gather(x_view, tables_flat):
    mesh = plsc.VectorSubcoreMesh(core_axis_name="c", subcore_axis_name="s")

    @functools.partial(
        pl.kernel,
        mesh=mesh,
        out_type=jax.ShapeDtypeStruct((_BATCH * _NCOL, _EMB), jnp.float32),
        compiler_params=pltpu.CompilerParams(use_tc_tiling_on_sc=False),
        scratch_types=[
            pltpu.VMEM((_NCHUNK, _CHUNK), jnp.int32),
            pltpu.VMEM((_ROWS_PER_W, _EMB), jnp.float32),
            pltpu.SemaphoreType.DMA,
        ],
    )
    def k(x_hbm, tab_hbm, out_hbm, idx_v, rows_v, sem):
        wid = lax.axis_index("s") * 2 + lax.axis_index("c")

        pltpu.sync_copy(x_hbm.at[wid], idx_v)

        def off_row(r, carry):
            row = idx_v.at[r]

            def off_vec(v, c2):
                p = r * _CHUNK + v * _LANES + lax.iota(jnp.int32, _LANES)
                col = lax.rem(p, _NCOL)
                off = jnp.maximum(col - 1, 0) * _VOCAB
                row[pl.ds(v * _LANES, _LANES)] = (
                    row[pl.ds(v * _LANES, _LANES)] + off)
                return c2

            return lax.fori_loop(0, _CHUNK // _LANES, off_vec, carry)

        lax.fori_loop(0, _NCHUNK, off_row, 0)

        def fire(c, carry):
            pltpu.async_copy(
                tab_hbm.at[idx_v.at[c]],
                rows_v.at[pl.ds(c * _CHUNK, _CHUNK)],
                sem,
            )
            return carry

        lax.fori_loop(0, _NCHUNK, fire, 0)

        pltpu.make_async_copy(
            tab_hbm.at[pl.ds(0, _ROWS_PER_W)], rows_v, sem).wait()

        pltpu.sync_copy(
            rows_v, out_hbm.at[pl.ds(wid * _ROWS_PER_W, _ROWS_PER_W)])

    return k(x_view, tables_flat)


def kernel(x, tables):
    x_view = x.astype(jnp.int32).reshape(_NW, _NCHUNK, _CHUNK)
    tables_flat = tables.reshape(_NUM_TABLES * _VOCAB, _EMB)
    out = _sc_gather(x_view, tables_flat)
    return out.reshape(_BATCH, _NCOL * _EMB)


# speedup vs baseline: 1.0004x; 1.0004x over previous
import functools

import jax
import jax.numpy as jnp
from jax import lax
from jax.experimental import pallas as pl
from jax.experimental.pallas import tpu as pltpu
from jax.experimental.pallas import tpu_sc as plsc

_NUM_TABLES = 26
_VOCAB = 100000
_EMB = 32
_BATCH = 4096
_NCOL = _NUM_TABLES + 1
_NW = 32
_CHUNK = _BATCH // _NW
_ROWS_PER_W = _CHUNK * _NCOL
_LANES = 16


def _sc_gather(x_t, tables):
    mesh = plsc.VectorSubcoreMesh(core_axis_name="c", subcore_axis_name="s")

    @functools.partial(
        pl.kernel,
        mesh=mesh,
        out_type=jax.ShapeDtypeStruct((_BATCH * _NCOL, _EMB), jnp.float32),
        compiler_params=pltpu.CompilerParams(use_tc_tiling_on_sc=False),
        scratch_types=[
            pltpu.VMEM((_NCOL, _CHUNK), jnp.int32),
            pltpu.VMEM((_NCOL, _CHUNK), jnp.int32),
            pltpu.VMEM((_ROWS_PER_W, _EMB), jnp.float32),
            pltpu.SemaphoreType.DMA,
            pltpu.SemaphoreType.DMA,
            pltpu.SemaphoreType.DMA,
        ],
    )
    def k(x_hbm, tab_hbm, out_hbm, idx_v, oidx_v, rows_v, isem, gsem, ssem):
        wid = lax.axis_index("s") * 2 + lax.axis_index("c")
        b0 = wid * _CHUNK

        pltpu.async_copy(x_hbm.at[:, pl.ds(b0, _CHUNK)], idx_v, isem)

        def orow(j, carry):
            row = oidx_v.at[j]

            def ovec(v, c2):
                i = v * _LANES + lax.iota(jnp.int32, _LANES)
                row[pl.ds(v * _LANES, _LANES)] = (b0 + i) * _NCOL + j
                return c2

            return lax.fori_loop(0, _CHUNK // _LANES, ovec, carry)

        lax.fori_loop(0, _NCOL, orow, 0)

        pltpu.make_async_copy(
            x_hbm.at[:, pl.ds(b0, _CHUNK)], idx_v, isem).wait()

        def fire_g(j, carry):
            t = jnp.maximum(j - 1, 0)
            pltpu.async_copy(
                tab_hbm.at[t].at[idx_v.at[j]],
                rows_v.at[pl.ds(j * _CHUNK, _CHUNK)],
                gsem,
            )
            return carry

        lax.fori_loop(0, _NCOL, fire_g, 0)

        pltpu.make_async_copy(
            tab_hbm.at[0].at[pl.ds(0, _ROWS_PER_W)], rows_v, gsem).wait()

        def fire_s(j, carry):
            pltpu.async_copy(
                rows_v.at[pl.ds(j * _CHUNK, _CHUNK)],
                out_hbm.at[oidx_v.at[j]],
                ssem,
            )
            return carry

        lax.fori_loop(0, _NCOL, fire_s, 0)

        pltpu.make_async_copy(
            rows_v, out_hbm.at[pl.ds(wid * _ROWS_PER_W, _ROWS_PER_W)],
            ssem).wait()

    return k(x_t, tables)


def kernel(x, tables):
    x_t = x.astype(jnp.int32).T
    out = _sc_gather(x_t, tables)
    return out.reshape(_BATCH, _NCOL * _EMB)
